# baseline (speedup 1.0000x reference)
import jax
import jax.numpy as jnp
from jax.experimental import pallas as pl
from jax.experimental.pallas import tpu as pltpu


def _body(pos_ref, hid_ref, mask_ref, emb_ref, out_ref, mask_out_ref):
    nct = emb_ref.shape[1]
    out_ref[0, :nct] = emb_ref[0]
    out_ref[0, nct:] = hid_ref[0]
    mask_out_ref[0, 0, :nct] = jnp.ones((nct,), mask_out_ref.dtype)
    mask_out_ref[0, 0, nct:] = mask_ref[0, 0]


def kernel(hidden_states, attention_mask, embeddings, position):
    B, S, D = hidden_states.shape
    _, NCT, _ = embeddings.shape
    pos = jnp.asarray(position, jnp.int32).reshape((1,))
    hid4 = hidden_states.reshape(B, S, 8, D // 8)
    emb4 = embeddings.reshape(embeddings.shape[0], NCT, 8, D // 8)
    mask3 = attention_mask.reshape(B, 1, S)

    grid_spec = pltpu.PrefetchScalarGridSpec(
        num_scalar_prefetch=1,
        grid=(B,),
        in_specs=[
            pl.BlockSpec((1, S, 8, D // 8), lambda b, p: (b, 0, 0, 0)),
            pl.BlockSpec((1, 1, S), lambda b, p: (b, 0, 0)),
            pl.BlockSpec((1, NCT, 8, D // 8), lambda b, p: (p[0], 0, 0, 0)),
        ],
        out_specs=[
            pl.BlockSpec((1, NCT + S, 8, D // 8), lambda b, p: (b, 0, 0, 0)),
            pl.BlockSpec((1, 1, NCT + S), lambda b, p: (b, 0, 0)),
        ],
    )

    out_hid, out_mask = pl.pallas_call(
        _body,
        grid_spec=grid_spec,
        out_shape=[
            jax.ShapeDtypeStruct((B, NCT + S, 8, D // 8), hidden_states.dtype),
            jax.ShapeDtypeStruct((B, 1, NCT + S), attention_mask.dtype),
        ],
    )(pos, hid4, mask3, emb4)
    return (out_hid.reshape(B, NCT + S, D), out_mask.reshape(B, NCT + S))
